# baseline (speedup 1.0000x reference)
import functools

import jax
import jax.numpy as jnp
from jax import lax
from jax.experimental import pallas as pl
from jax.experimental.pallas import tpu as pltpu
from jax.experimental.pallas import tpu_sc as plsc

NC = 2
NS = 16
L = 16
G1 = 128
PBITS = 10
BB = 64


def _sc_body(nb, sl_len, h, wid_hbm, sp_hbm, wtab_hbm, satab_hbm,
             ptab_hbm, out_hbm,
             sa_v, po_v, wid_v, sp_v, rows_v, outb_v,
             isem, gsem, osem):
    w = lax.axis_index("s") * NC + lax.axis_index("c")
    rows_per_tile = nb // (NC * NS)
    base = w * rows_per_tile
    nh = h // L
    g2 = sl_len - G1
    half = sl_len // 2

    pltpu.sync_copy(satab_hbm, sa_v)
    pltpu.sync_copy(ptab_hbm, po_v)

    def issue_idx(ci, sl):
        pltpu.async_copy(wid_hbm.at[base + ci], wid_v.at[sl], isem.at[sl])
        pltpu.async_copy(sp_hbm.at[base + ci], sp_v.at[sl], isem.at[sl])

    def wait_idx(sl):
        pltpu.make_async_copy(wid_hbm.at[0], wid_v.at[sl], isem.at[sl]).wait()
        pltpu.make_async_copy(sp_hbm.at[0], sp_v.at[sl], isem.at[sl]).wait()

    def issue_gather(sl):
        pltpu.async_copy(wtab_hbm.at[wid_v.at[sl, pl.ds(0, G1)]],
                         rows_v.at[sl, pl.ds(0, G1)], gsem.at[sl])
        pltpu.async_copy(wtab_hbm.at[wid_v.at[sl, pl.ds(G1, g2)]],
                         rows_v.at[sl, pl.ds(G1, g2)], gsem.at[sl])

    def wait_gather(sl):
        pltpu.make_async_copy(wtab_hbm.at[wid_v.at[sl, pl.ds(0, G1)]],
                              rows_v.at[sl, pl.ds(0, G1)], gsem.at[sl]).wait()
        pltpu.make_async_copy(wtab_hbm.at[wid_v.at[sl, pl.ds(G1, g2)]],
                              rows_v.at[sl, pl.ds(G1, g2)], gsem.at[sl]).wait()

    def issue_out(ci, sl):
        pltpu.async_copy(outb_v.at[sl],
                         out_hbm.at[pl.ds((base + ci) * half, half)],
                         osem.at[sl])

    def wait_out(sl):
        pltpu.make_async_copy(outb_v.at[sl], out_hbm.at[pl.ds(0, half)],
                              osem.at[sl]).wait()

    issue_idx(0, 0)
    wait_idx(0)
    issue_gather(0)
    issue_idx(1, 1)

    def chunk_step(ci, sl):
        other = 1 - sl
        wait_gather(sl)

        @pl.when(ci + 1 < rows_per_tile)
        def _():
            wait_idx(other)
            issue_gather(other)

        @pl.when(ci >= 2)
        def _():
            wait_out(sl)

        def process8(sa_i, p_i, lane_base, t_base):
            for j8 in range(8):
                t = t_base + j8
                sj = sa_i[lane_base + j8]
                pj = p_i[lane_base + j8]
                row = t_base // 2 + j8 // 2
                col = (j8 % 2) * h
                for k in range(nh):
                    outb_v[sl, row, pl.ds(col + k * L, L)] = (
                        rows_v[sl, t, pl.ds(k * L, L)]
                        + sa_v[sj, pl.ds(k * L, L)]
                        + po_v[pj, pl.ds(k * L, L)])

        def unpack_ids(t0):
            sp = sp_v[sl, pl.ds(t0, L)]
            sa_i = lax.shift_right_logical(sp, PBITS)
            p_i = lax.bitwise_and(sp, jnp.int32((1 << PBITS) - 1))
            return sa_i, p_i

        def tb_body(tb, inner):
            t0 = tb * L
            sa_i, p_i = unpack_ids(t0)
            process8(sa_i, p_i, 0, t0)
            process8(sa_i, p_i, 8, t0 + 8)
            return inner

        lax.fori_loop(0, sl_len // L, tb_body, 0)
        if sl_len % L:
            sa_i, p_i = unpack_ids(sl_len - L)
            process8(sa_i, p_i, 8, sl_len - 8)
        issue_out(ci, sl)

        @pl.when(ci + 2 < rows_per_tile)
        def _():
            issue_idx(ci + 2, sl)

    def chunk_pair(cp, carry):
        chunk_step(cp * 2, 0)
        chunk_step(cp * 2 + 1, 1)
        return carry

    lax.fori_loop(0, rows_per_tile // 2, chunk_pair, 0)
    wait_out(0)
    wait_out(1)


def _tc_ln_body(s, h, g_ref, b_ref, x_ref, o_ref):
    x = x_ref[...]
    g = g_ref[...]
    bb = b_ref[...]
    outs = []
    for piece in (x[:, :h], x[:, h:]):
        mean = jnp.mean(piece, axis=1, keepdims=True)
        var = jnp.mean(piece * piece, axis=1, keepdims=True) - mean * mean
        r = lax.rsqrt(var + 1e-12)
        outs.append((piece - mean) * r * g + bb)
    y = jnp.stack(outs, axis=1)
    o_ref[...] = y.reshape(BB, s, h)


def kernel(input_ids, age_ids, seg_ids, posi_ids, word_table, seg_table,
           age_table, posi_table, ln_gamma, ln_beta):
    b, s = input_ids.shape
    _, h = word_table.shape
    n_seg = seg_table.shape[0]
    n_age = age_table.shape[0]
    n_pos = posi_table.shape[0]
    assert h % L == 0 and n_pos <= (1 << PBITS)
    assert b % (NC * NS) == 0 and (b // (NC * NS)) % 2 == 0
    assert b // (NC * NS) >= 4 and b % BB == 0
    assert s % 8 == 0 and G1 < s <= 2 * G1

    wids = input_ids.astype(jnp.int32)
    sp = ((seg_ids * n_age + age_ids) * (1 << PBITS) + posi_ids).astype(jnp.int32)
    satab = (seg_table[:, None, :] + age_table[None, :, :]).reshape(n_seg * n_age, h)

    sc_fn = pl.kernel(
        functools.partial(_sc_body, b, s, h),
        out_type=jax.ShapeDtypeStruct((b * s // 2, 2 * h), jnp.float32),
        mesh=plsc.VectorSubcoreMesh(core_axis_name="c", subcore_axis_name="s",
                                    num_cores=NC, num_subcores=NS),
        compiler_params=pltpu.CompilerParams(use_tc_tiling_on_sc=False,
                                             needs_layout_passes=False),
        scratch_types=[
            pltpu.VMEM((n_seg * n_age, h), jnp.float32),
            pltpu.VMEM((n_pos, h), jnp.float32),
            pltpu.VMEM((2, s), jnp.int32),
            pltpu.VMEM((2, s), jnp.int32),
            pltpu.VMEM((2, s, h), jnp.float32),
            pltpu.VMEM((2, s // 2, 2 * h), jnp.float32),
            pltpu.SemaphoreType.DMA((2,)),
            pltpu.SemaphoreType.DMA((2,)),
            pltpu.SemaphoreType.DMA((2,)),
        ],
    )
    sums = sc_fn(wids, sp, word_table, satab, posi_table)

    tc_fn = pl.pallas_call(
        functools.partial(_tc_ln_body, s, h),
        grid=(b // BB,),
        in_specs=[
            pl.BlockSpec((h,), lambda i: (0,)),
            pl.BlockSpec((h,), lambda i: (0,)),
            pl.BlockSpec((BB * s // 2, 2 * h), lambda i: (i, 0)),
        ],
        out_specs=pl.BlockSpec((BB, s, h), lambda i: (i, 0, 0)),
        out_shape=jax.ShapeDtypeStruct((b, s, h), jnp.float32),
        compiler_params=pltpu.CompilerParams(
            dimension_semantics=("parallel",)),
    )
    return tc_fn(ln_gamma, ln_beta, sums)


# speedup vs baseline: 1.8462x; 1.8462x over previous
import functools

import jax
import jax.numpy as jnp
from jax import lax
from jax.experimental import pallas as pl
from jax.experimental.pallas import tpu as pltpu
from jax.experimental.pallas import tpu_sc as plsc

NC = 2
NS = 16
L = 16
G1 = 128
PBITS = 10


def _rsqrt(x):
    i = plsc.bitcast(x, jnp.int32)
    i = jnp.int32(0x5F3759DF) - lax.shift_right_logical(i, 1)
    y = plsc.bitcast(i, jnp.float32)
    for _ in range(1):
        y = y * (1.5 - 0.5 * x * y * y)
    return y


def _bcast(x, lane):
    idx = jnp.full((L,), lane, jnp.int32)
    dnums = lax.GatherDimensionNumbers(
        offset_dims=(), collapsed_slice_dims=(0,), start_index_map=(0,))
    return lax.gather(x, idx[:, None], dnums, (1,),
                      mode=lax.GatherScatterMode.PROMISE_IN_BOUNDS)


def _sc_body(nb, sl_len, h, wid_hbm, sp_hbm, wtab_hbm, satab_hbm,
             ptab_hbm, g_hbm, b_hbm, out_hbm,
             sa_v, po_v, ga_v, be_v, wid_v, sp_v, rows_v, outb_v,
             isem, gsem, osem):
    w = lax.axis_index("s") * NC + lax.axis_index("c")
    rows_per_tile = nb // (NC * NS)
    base = w * rows_per_tile
    nh = h // L
    g2 = sl_len - G1

    pltpu.sync_copy(satab_hbm, sa_v)
    pltpu.sync_copy(ptab_hbm, po_v)
    pltpu.sync_copy(g_hbm, ga_v)
    pltpu.sync_copy(b_hbm, be_v)
    gs = [ga_v[pl.ds(k * L, L)] for k in range(nh)]
    bs = [be_v[pl.ds(k * L, L)] for k in range(nh)]
    inv_h = 1.0 / h
    iota = lax.iota(jnp.int32, L)

    def issue_idx(ci, sl):
        off = (base + ci) * sl_len
        pltpu.async_copy(wid_hbm.at[pl.ds(off, sl_len)], wid_v.at[sl],
                         isem.at[sl])
        pltpu.async_copy(sp_hbm.at[pl.ds(off, sl_len)], sp_v.at[sl],
                         isem.at[sl])

    def wait_idx(sl):
        pltpu.make_async_copy(wid_hbm.at[pl.ds(0, sl_len)], wid_v.at[sl],
                              isem.at[sl]).wait()
        pltpu.make_async_copy(sp_hbm.at[pl.ds(0, sl_len)], sp_v.at[sl],
                              isem.at[sl]).wait()

    def issue_gather(sl):
        pltpu.async_copy(wtab_hbm.at[wid_v.at[sl, pl.ds(0, G1)]],
                         rows_v.at[sl, pl.ds(0, G1)], gsem.at[sl])
        pltpu.async_copy(wtab_hbm.at[wid_v.at[sl, pl.ds(G1, g2)]],
                         rows_v.at[sl, pl.ds(G1, g2)], gsem.at[sl])

    def wait_gather(sl):
        pltpu.make_async_copy(wtab_hbm.at[wid_v.at[sl, pl.ds(0, G1)]],
                              rows_v.at[sl, pl.ds(0, G1)], gsem.at[sl]).wait()
        pltpu.make_async_copy(wtab_hbm.at[wid_v.at[sl, pl.ds(G1, g2)]],
                              rows_v.at[sl, pl.ds(G1, g2)], gsem.at[sl]).wait()

    def issue_out(ci, sl):
        pltpu.async_copy(outb_v.at[sl], out_hbm.at[base + ci], osem.at[sl])

    def wait_out(sl):
        pltpu.make_async_copy(outb_v.at[sl], out_hbm.at[base], osem.at[sl]).wait()

    issue_idx(0, 0)
    wait_idx(0)
    issue_gather(0)
    issue_idx(1, 1)

    def chunk_step(ci, sl):
        other = 1 - sl
        wait_gather(sl)

        @pl.when(ci + 1 < rows_per_tile)
        def _():
            wait_idx(other)
            issue_gather(other)

        @pl.when(ci >= 2)
        def _():
            wait_out(sl)

        def process8(sa_i, p_i, lane_base, t_base):
            vss = []
            s1a = s2a = None
            for j8 in range(8):
                t = t_base + j8
                sj = sa_i[lane_base + j8]
                pj = p_i[lane_base + j8]
                vs = [rows_v[sl, t, pl.ds(k * L, L)]
                      + sa_v[sj, pl.ds(k * L, L)]
                      + po_v[pj, pl.ds(k * L, L)]
                      for k in range(nh)]
                s1 = vs[0]
                for v in vs[1:]:
                    s1 = s1 + v
                s2 = vs[0] * vs[0]
                for v in vs[1:]:
                    s2 = s2 + v * v
                tot = _bcast(plsc.cumsum(s1), L - 1)
                totq = _bcast(plsc.cumsum(s2), L - 1)
                if j8 == 0:
                    s1a, s2a = tot, totq
                else:
                    lane = iota == j8
                    s1a = jnp.where(lane, tot, s1a)
                    s2a = jnp.where(lane, totq, s2a)
                vss.append(vs)
            mean_v = s1a * inv_h
            var_v = s2a * inv_h - mean_v * mean_v
            r_v = _rsqrt(var_v + 1e-12)
            for j8 in range(8):
                t = t_base + j8
                mj = _bcast(mean_v, j8)
                rj = _bcast(r_v, j8)
                for k in range(nh):
                    outb_v[sl, t, pl.ds(k * L, L)] = \
                        (vss[j8][k] - mj) * rj * gs[k] + bs[k]

        def unpack_ids(t0):
            sp = sp_v[sl, pl.ds(t0, L)]
            sa_i = lax.shift_right_logical(sp, PBITS)
            p_i = lax.bitwise_and(sp, jnp.int32((1 << PBITS) - 1))
            return sa_i, p_i

        def tb_body(tb, inner):
            t0 = tb * L
            sa_i, p_i = unpack_ids(t0)
            process8(sa_i, p_i, 0, t0)
            process8(sa_i, p_i, 8, t0 + 8)
            return inner

        lax.fori_loop(0, sl_len // L, tb_body, 0)
        if sl_len % L:
            sa_i, p_i = unpack_ids(sl_len - L)
            process8(sa_i, p_i, 8, sl_len - 8)
        issue_out(ci, sl)

        @pl.when(ci + 2 < rows_per_tile)
        def _():
            issue_idx(ci + 2, sl)

    def chunk_pair(cp, carry):
        chunk_step(cp * 2, 0)
        chunk_step(cp * 2 + 1, 1)
        return carry

    lax.fori_loop(0, rows_per_tile // 2, chunk_pair, 0)
    wait_out(0)
    wait_out(1)


def kernel(input_ids, age_ids, seg_ids, posi_ids, word_table, seg_table,
           age_table, posi_table, ln_gamma, ln_beta):
    b, s = input_ids.shape
    _, h = word_table.shape
    n_seg = seg_table.shape[0]
    n_age = age_table.shape[0]
    n_pos = posi_table.shape[0]
    n_tok = b * s
    assert h % L == 0 and n_pos <= (1 << PBITS)
    assert b % (NC * NS) == 0 and (b // (NC * NS)) % 2 == 0
    assert b // (NC * NS) >= 4
    assert s % 8 == 0 and G1 < s <= 2 * G1 and (s * 4) % 8 == 0

    wids = input_ids.reshape(n_tok).astype(jnp.int32)
    sp = ((seg_ids.reshape(n_tok) * n_age + age_ids.reshape(n_tok)) * (1 << PBITS)
          + posi_ids.reshape(n_tok)).astype(jnp.int32)
    satab = (seg_table[:, None, :] + age_table[None, :, :]).reshape(n_seg * n_age, h)

    fn = pl.kernel(
        functools.partial(_sc_body, b, s, h),
        out_type=jax.ShapeDtypeStruct((b, s, h), jnp.float32),
        mesh=plsc.VectorSubcoreMesh(core_axis_name="c", subcore_axis_name="s",
                                    num_cores=NC, num_subcores=NS),
        compiler_params=pltpu.CompilerParams(use_tc_tiling_on_sc=False,
                                             needs_layout_passes=False),
        scratch_types=[
            pltpu.VMEM((n_seg * n_age, h), jnp.float32),
            pltpu.VMEM((n_pos, h), jnp.float32),
            pltpu.VMEM((h,), jnp.float32),
            pltpu.VMEM((h,), jnp.float32),
            pltpu.VMEM((2, s), jnp.int32),
            pltpu.VMEM((2, s), jnp.int32),
            pltpu.VMEM((2, s, h), jnp.float32),
            pltpu.VMEM((2, s, h), jnp.float32),
            pltpu.SemaphoreType.DMA((2,)),
            pltpu.SemaphoreType.DMA((2,)),
            pltpu.SemaphoreType.DMA((2,)),
        ],
    )
    return fn(wids, sp, word_table, satab, posi_table, ln_gamma, ln_beta)
